# Initial kernel scaffold; baseline (speedup 1.0000x reference)
import jax
import jax.numpy as jnp
from jax.experimental import pallas as pl


def kernel(idx, targets, word_embs):
    raise NotImplementedError("write your pallas kernel here")


# speedup vs baseline: 1.0142x; 1.0142x over previous
import functools

import jax
import jax.numpy as jnp
from jax import lax
from jax.experimental import pallas as pl
from jax.experimental.pallas import tpu as pltpu
from jax.experimental.pallas import tpu_sc as plsc

DICT_SIZE = 100000
EMB = 128
BATCH = 16384

NUM_CORES = 2
NUM_SUBCORES = 16
NUM_WORKERS = NUM_CORES * NUM_SUBCORES
BPW = BATCH // NUM_WORKERS
CHUNK = 256
NCHUNKS = BPW // CHUNK
GROUPS = CHUNK // 16
LANES = 16
VPR = EMB // LANES

_mesh = plsc.VectorSubcoreMesh(core_axis_name="c", subcore_axis_name="s")


@functools.partial(
    pl.kernel,
    out_type=jax.ShapeDtypeStruct((BATCH,), jnp.float32),
    mesh=_mesh,
    compiler_params=pltpu.CompilerParams(needs_layout_passes=False),
    scratch_types=[
        pltpu.VMEM((BPW,), jnp.int32),
        pltpu.VMEM((BPW,), jnp.int32),
        pltpu.VMEM((CHUNK, EMB), jnp.float32),
        pltpu.VMEM((CHUNK, EMB), jnp.float32),
        pltpu.VMEM((BPW,), jnp.float32),
        pltpu.SemaphoreType.DMA,
    ],
)
def _word_sim(idx_hbm, tgt_hbm, table_hbm, out_hbm,
              idx_v, tgt_v, xs_v, ys_v, out_v, sem):
    wid = lax.axis_index("s") * NUM_CORES + lax.axis_index("c")
    base = wid * BPW
    pltpu.sync_copy(idx_hbm.at[pl.ds(base, BPW)], idx_v)
    pltpu.sync_copy(tgt_hbm.at[pl.ds(base, BPW)], tgt_v)

    lane = lax.broadcasted_iota(jnp.int32, (16,), 0)

    for c in range(NCHUNKS):
        cp_x = pltpu.async_copy(
            table_hbm.at[idx_v.at[pl.ds(c * CHUNK, CHUNK)]], xs_v, sem)
        cp_y = pltpu.async_copy(
            table_hbm.at[tgt_v.at[pl.ds(c * CHUNK, CHUNK)]], ys_v, sem)
        cp_x.wait()
        cp_y.wait()

        def group_body(g, carry, c=c):
            row0 = g * LANES
            res = jnp.zeros((LANES,), jnp.float32)
            for r in range(LANES):
                parts = [xs_v[row0 + r, pl.ds(k * LANES, LANES)]
                         * ys_v[row0 + r, pl.ds(k * LANES, LANES)]
                         for k in range(VPR)]
                while len(parts) > 1:
                    parts = [parts[i] + parts[i + 1]
                             for i in range(0, len(parts) - 1, 2)] \
                        + ([parts[-1]] if len(parts) % 2 else [])
                s = jnp.sum(parts[0])
                res = jnp.where(lane == r, s, res)
            out_v[pl.ds(c * CHUNK + row0, LANES)] = res
            return carry

        lax.fori_loop(0, GROUPS, group_body, 0)

    pltpu.sync_copy(out_v, out_hbm.at[pl.ds(base, BPW)])


def kernel(idx, targets, word_embs):
    return _word_sim(idx, targets, word_embs)


# speedup vs baseline: 1.3177x; 1.2992x over previous
import functools

import jax
import jax.numpy as jnp
from jax import lax
from jax.experimental import pallas as pl
from jax.experimental.pallas import tpu as pltpu
from jax.experimental.pallas import tpu_sc as plsc

DICT_SIZE = 100000
EMB = 128
BATCH = 16384

NUM_CORES = 2
NUM_SUBCORES = 16
NUM_WORKERS = NUM_CORES * NUM_SUBCORES
BPW = BATCH // NUM_WORKERS
CHUNK = 128
NCHUNKS = BPW // CHUNK
GROUPS = CHUNK // 16
LANES = 16
VPR = EMB // LANES

_mesh = plsc.VectorSubcoreMesh(core_axis_name="c", subcore_axis_name="s")


def _tree_add(parts):
    while len(parts) > 1:
        parts = [parts[i] + parts[i + 1]
                 for i in range(0, len(parts) - 1, 2)] \
            + ([parts[-1]] if len(parts) % 2 else [])
    return parts[0]


@functools.partial(
    pl.kernel,
    out_type=jax.ShapeDtypeStruct((BATCH,), jnp.float32),
    mesh=_mesh,
    compiler_params=pltpu.CompilerParams(needs_layout_passes=False),
    scratch_types=[
        pltpu.VMEM((BPW,), jnp.int32),
        pltpu.VMEM((BPW,), jnp.int32),
        pltpu.VMEM((2, CHUNK, EMB), jnp.float32),
        pltpu.VMEM((2, CHUNK, EMB), jnp.float32),
        pltpu.VMEM((LANES * LANES,), jnp.float32),
        pltpu.VMEM((BPW,), jnp.float32),
        pltpu.SemaphoreType.DMA,
        pltpu.SemaphoreType.DMA,
    ],
)
def _word_sim(idx_hbm, tgt_hbm, table_hbm, out_hbm,
              idx_v, tgt_v, xs_v, ys_v, tp_v, out_v, sem0, sem1):
    wid = lax.axis_index("s") * NUM_CORES + lax.axis_index("c")
    base = wid * BPW
    cp_i = pltpu.async_copy(idx_hbm.at[pl.ds(base, BPW)], idx_v, sem0)
    cp_t = pltpu.async_copy(tgt_hbm.at[pl.ds(base, BPW)], tgt_v, sem1)
    cp_i.wait()
    cp_t.wait()

    lane = lax.broadcasted_iota(jnp.int32, (LANES,), 0)
    sems = (sem0, sem1)

    def fire(c):
        buf = c % 2
        cpx = pltpu.async_copy(
            table_hbm.at[idx_v.at[pl.ds(c * CHUNK, CHUNK)]],
            xs_v.at[buf], sems[buf])
        cpy = pltpu.async_copy(
            table_hbm.at[tgt_v.at[pl.ds(c * CHUNK, CHUNK)]],
            ys_v.at[buf], sems[buf])
        return cpx, cpy

    pending = fire(0)
    for c in range(NCHUNKS):
        nxt = fire(c + 1) if c + 1 < NCHUNKS else None
        pending[0].wait()
        pending[1].wait()
        buf = c % 2

        def group_body(g, carry, c=c, buf=buf):
            row0 = g * LANES
            for r in range(LANES):
                p = _tree_add(
                    [xs_v[buf, row0 + r, pl.ds(k * LANES, LANES)]
                     * ys_v[buf, row0 + r, pl.ds(k * LANES, LANES)]
                     for k in range(VPR)])
                plsc.store_scatter(tp_v, [lane * LANES + r], p)
            res = _tree_add(
                [tp_v[pl.ds(j * LANES, LANES)] for j in range(LANES)])
            out_v[pl.ds(c * CHUNK + row0, LANES)] = res
            return carry

        lax.fori_loop(0, GROUPS, group_body, 0)
        pending = nxt

    pltpu.sync_copy(out_v, out_hbm.at[pl.ds(base, BPW)])


def kernel(idx, targets, word_embs):
    return _word_sim(idx, targets, word_embs)
